# baseline (speedup 1.0000x reference)
import functools
import math

import jax
import jax.numpy as jnp
from jax import lax
from jax.experimental import pallas as pl
from jax.experimental.pallas import tpu as pltpu
from jax.experimental.pallas import tpu_sc as plsc

SMOOTH = 0.1
CONF = 1.0 - SMOOTH

_NC = 2
_NS = 16
_NW = _NC * _NS
_LANES = 16


def _sc_body(n_rows, v, t_len, x_hbm, tgt_hbm, m_hbm, out_hbm,
             tgt_v, m_v, gbuf, acc_out, sem):
    per_w = n_rows // _NW
    wpb = t_len // per_w
    wid = lax.axis_index("s") * _NC + lax.axis_index("c")
    rbase = wid * per_w
    bi = wid // wpb
    toff = (wid % wpb) * per_w

    pltpu.sync_copy(tgt_hbm.at[bi, pl.ds(toff, per_w)], tgt_v)
    pltpu.sync_copy(m_hbm.at[bi, pl.ds(toff, per_w)], m_v)

    nvec = per_w // _LANES
    tregs = [tgt_v[pl.ds(c * _LANES, _LANES)] for c in range(nvec)]

    handles = []
    for r in range(per_w):
        t_r = tregs[r // _LANES][r % _LANES]
        off = pl.multiple_of(t_r & ~(_LANES - 1), _LANES)
        handles.append(pltpu.async_copy(
            x_hbm.at[rbase + r, pl.ds(off, _LANES)],
            gbuf.at[pl.ds(r * _LANES, _LANES)], sem))
    for h in handles:
        h.wait()

    lane = lax.broadcasted_iota(jnp.int32, (_LANES,), 0)
    g16 = jnp.zeros((_LANES,), jnp.float32)
    ms16 = jnp.zeros((_LANES,), jnp.float32)
    for c in range(nvec):
        ms16 = ms16 + m_v[pl.ds(c * _LANES, _LANES)]
    mregs = [m_v[pl.ds(c * _LANES, _LANES)] for c in range(nvec)]
    for r in range(per_w):
        t_r = tregs[r // _LANES][r % _LANES]
        m_r = mregs[r // _LANES][r % _LANES]
        gv16 = gbuf[pl.ds(r * _LANES, _LANES)]
        eq = lane == (t_r & (_LANES - 1))
        g16 = g16 + jnp.where(eq, gv16 * m_r, 0.0)

    acc_out[pl.ds(0, _LANES)] = g16
    acc_out[pl.ds(_LANES, _LANES)] = ms16
    pltpu.sync_copy(acc_out, out_hbm.at[wid])


def _make_sc(n_rows, v, t_len):
    per_w = n_rows // _NW
    mesh = plsc.VectorSubcoreMesh(core_axis_name="c", subcore_axis_name="s")
    return pl.kernel(
        functools.partial(_sc_body, n_rows, v, t_len),
        out_type=jax.ShapeDtypeStruct((_NW, 2 * _LANES), jnp.float32),
        mesh=mesh,
        scratch_types=[
            pltpu.VMEM((per_w,), jnp.int32),
            pltpu.VMEM((per_w,), jnp.float32),
            pltpu.VMEM((per_w * _LANES,), jnp.float32),
            pltpu.VMEM((2 * _LANES,), jnp.float32),
            pltpu.SemaphoreType.DMA,
        ],
        compiler_params=pltpu.CompilerParams(use_tc_tiling_on_sc=True),
    )


def _tc_body(c_const, eps, x_ref, m_ref, sc_ref, out_ref, acc_s):
    i = pl.program_id(0)

    @pl.when(i == 0)
    def _init():
        acc_s[0, 0] = 0.0

    x = x_ref[...]
    m = m_ref[...]
    rowsum = jnp.sum(x, axis=1, keepdims=True)
    acc_s[0, 0] += jnp.sum(rowsum * m)

    @pl.when(i == pl.num_programs(0) - 1)
    def _fin():
        sc = sc_ref[...]
        g_tot = jnp.sum(sc[:, :_LANES])
        mt = jnp.sum(sc[:, _LANES:])
        out_ref[0, 0] = (c_const * mt - eps * acc_s[0, 0]
                         + (eps - CONF) * g_tot) / mt


def _make_tc(n_rows, v, block_r):
    ni = n_rows // block_r
    eps = SMOOTH / (v - 1)
    c_const = (v - 1) * eps * math.log(eps) + CONF * math.log(CONF)
    return pl.pallas_call(
        functools.partial(_tc_body, c_const, eps),
        grid=(ni,),
        in_specs=[
            pl.BlockSpec((block_r, v), lambda i: (i, 0)),
            pl.BlockSpec((block_r, 1), lambda i: (i, 0)),
            pl.BlockSpec((_NW, 2 * _LANES), lambda i: (0, 0)),
        ],
        out_specs=pl.BlockSpec((1, 1), lambda i: (0, 0),
                               memory_space=pltpu.SMEM),
        out_shape=jax.ShapeDtypeStruct((1, 1), jnp.float32),
        scratch_shapes=[
            pltpu.SMEM((1, 1), jnp.float32),
        ],
    )


def kernel(input, target, mask):
    b, t, v = input.shape
    n = b * t
    x = input.reshape(n, v)

    sc_part = _make_sc(n, v, t)(x, target.astype(jnp.int32), mask)
    return _make_tc(n, v, 128)(x, mask.reshape(n, 1), sc_part)[0, 0]


# speedup vs baseline: 1.0315x; 1.0315x over previous
import functools
import math

import jax
import jax.numpy as jnp
from jax import lax
from jax.experimental import pallas as pl
from jax.experimental.pallas import tpu as pltpu
from jax.experimental.pallas import tpu_sc as plsc

SMOOTH = 0.1
CONF = 1.0 - SMOOTH

_NC = 2
_NS = 16
_NW = _NC * _NS
_LANES = 16


def _sc_body(n_rows, v, x_hbm, tgt_hbm, m_hbm, out_hbm,
             tgt_v, m_v, gbuf, acc_out, sem):
    per_w = n_rows // _NW
    wid = lax.axis_index("s") * _NC + lax.axis_index("c")
    rbase = wid * per_w

    pltpu.sync_copy(tgt_hbm.at[pl.ds(rbase, per_w)], tgt_v)
    pltpu.sync_copy(m_hbm.at[pl.ds(rbase, per_w)], m_v)

    nvec = per_w // _LANES
    tregs = [tgt_v[pl.ds(c * _LANES, _LANES)] for c in range(nvec)]

    handles = []
    for r in range(per_w):
        t_r = tregs[r // _LANES][r % _LANES]
        off = pl.multiple_of(t_r & ~(_LANES - 1), _LANES)
        handles.append(pltpu.async_copy(
            x_hbm.at[rbase + r, pl.ds(off, _LANES)],
            gbuf.at[pl.ds(r * _LANES, _LANES)], sem))
    for h in handles:
        h.wait()

    lane = lax.broadcasted_iota(jnp.int32, (_LANES,), 0)
    g16 = jnp.zeros((_LANES,), jnp.float32)
    ms16 = jnp.zeros((_LANES,), jnp.float32)
    for c in range(nvec):
        ms16 = ms16 + m_v[pl.ds(c * _LANES, _LANES)]
    mregs = [m_v[pl.ds(c * _LANES, _LANES)] for c in range(nvec)]
    for r in range(per_w):
        t_r = tregs[r // _LANES][r % _LANES]
        m_r = mregs[r // _LANES][r % _LANES]
        gv16 = gbuf[pl.ds(r * _LANES, _LANES)]
        eq = lane == (t_r & (_LANES - 1))
        g16 = g16 + jnp.where(eq, gv16 * m_r, 0.0)

    acc_out[pl.ds(0, _LANES)] = g16
    acc_out[pl.ds(_LANES, _LANES)] = ms16
    pltpu.sync_copy(acc_out, out_hbm.at[wid])


def _make_sc(n_rows, v):
    per_w = n_rows // _NW
    mesh = plsc.VectorSubcoreMesh(core_axis_name="c", subcore_axis_name="s")
    return pl.kernel(
        functools.partial(_sc_body, n_rows, v),
        out_type=jax.ShapeDtypeStruct((_NW, 2 * _LANES), jnp.float32),
        mesh=mesh,
        scratch_types=[
            pltpu.VMEM((per_w,), jnp.int32),
            pltpu.VMEM((per_w,), jnp.float32),
            pltpu.VMEM((per_w * _LANES,), jnp.float32),
            pltpu.VMEM((2 * _LANES,), jnp.float32),
            pltpu.SemaphoreType.DMA,
        ],
        compiler_params=pltpu.CompilerParams(use_tc_tiling_on_sc=True),
    )


def _tc_body(c_const, eps, x_ref, m_ref, sc_ref, out_ref, acc_s):
    i = pl.program_id(0)

    @pl.when(i == 0)
    def _init():
        acc_s[0, 0] = 0.0

    x = x_ref[...]
    m = m_ref[...]
    rowsum = jnp.sum(x, axis=1, keepdims=True)
    acc_s[0, 0] += jnp.sum(rowsum * m)

    @pl.when(i == pl.num_programs(0) - 1)
    def _fin():
        sc = sc_ref[...]
        g_tot = jnp.sum(sc[:, :_LANES])
        mt = jnp.sum(sc[:, _LANES:])
        out_ref[0, 0] = (c_const * mt - eps * acc_s[0, 0]
                         + (eps - CONF) * g_tot) / mt


def _make_tc(n_rows, v, block_r):
    ni = n_rows // block_r
    eps = SMOOTH / (v - 1)
    c_const = (v - 1) * eps * math.log(eps) + CONF * math.log(CONF)
    return pl.pallas_call(
        functools.partial(_tc_body, c_const, eps),
        grid=(ni,),
        in_specs=[
            pl.BlockSpec((block_r, v), lambda i: (i, 0)),
            pl.BlockSpec((block_r, 1), lambda i: (i, 0)),
            pl.BlockSpec((_NW, 2 * _LANES), lambda i: (0, 0)),
        ],
        out_specs=pl.BlockSpec((1, 1), lambda i: (0, 0),
                               memory_space=pltpu.SMEM),
        out_shape=jax.ShapeDtypeStruct((1, 1), jnp.float32),
        scratch_shapes=[
            pltpu.SMEM((1, 1), jnp.float32),
        ],
    )


def kernel(input, target, mask):
    b, t, v = input.shape
    n = b * t
    x = input.reshape(n, v)
    tgt = target.reshape(n).astype(jnp.int32)
    m = mask.reshape(n)

    sc_part = _make_sc(n, v)(x, tgt, m)
    return _make_tc(n, v, 128)(x, m.reshape(n, 1), sc_part)[0, 0]
